# Initial kernel scaffold; baseline (speedup 1.0000x reference)
import jax
import jax.numpy as jnp
from jax.experimental import pallas as pl


def kernel(cls_pred, cls_label):
    raise NotImplementedError("write your pallas kernel here")


# speedup vs baseline: 3.3971x; 3.3971x over previous
import functools

import jax
import jax.numpy as jnp
from jax.experimental import pallas as pl
from jax.experimental.pallas import tpu as pltpu

ALPHA = 0.25
LOSS_WEIGHT = 1.0

BLOCK_ROWS = 4000


def _loss_block_kernel(x_ref, lbl_ref, out_ref):
    i = pl.program_id(0)
    x = x_ref[...]
    lbl = lbl_ref[0]

    ax = jnp.abs(x)
    t = jnp.exp(-ax)
    sp = jnp.maximum(x, 0.0) + jnp.log1p(t)
    u = 1.0 / (1.0 + t)
    p = jnp.where(x >= 0, u, 1.0 - u)
    f = sp * p * p

    cols = jax.lax.broadcasted_iota(jnp.int32, x.shape, 1)
    mask = (cols == lbl).astype(x.dtype)
    w = 0.75 - 0.5 * mask
    partial = jnp.sum(w * f, axis=(0, 1), keepdims=True)

    @pl.when(i == 0)
    def _():
        out_ref[...] = jnp.zeros_like(out_ref)

    out_ref[...] += partial


def kernel(cls_pred, cls_label):
    N, C = cls_pred.shape
    R = BLOCK_ROWS
    G = N // R
    lbl3 = cls_label.astype(jnp.int32).reshape(G, R, 1)

    total = pl.pallas_call(
        _loss_block_kernel,
        grid=(G,),
        in_specs=[
            pl.BlockSpec((R, C), lambda i: (i, 0)),
            pl.BlockSpec((1, R, 1), lambda i: (i, 0, 0)),
        ],
        out_specs=pl.BlockSpec((1, 1), lambda i: (0, 0)),
        out_shape=jax.ShapeDtypeStruct((1, 1), jnp.float32),
    )(cls_pred, lbl3)

    return (LOSS_WEIGHT / (N * C)) * total[0, 0]


# speedup vs baseline: 3.5992x; 1.0595x over previous
import functools

import jax
import jax.numpy as jnp
from jax.experimental import pallas as pl
from jax.experimental.pallas import tpu as pltpu

ALPHA = 0.25
LOSS_WEIGHT = 1.0

BLOCK_ROWS = 4000


def _loss_block_kernel(x_ref, lbl_ref, out_ref):
    i = pl.program_id(0)
    x = x_ref[...]
    lbl = lbl_ref[0]

    th = jnp.tanh(0.5 * x)
    sig = 0.5 + 0.5 * th
    m = 0.5 - 0.5 * th
    f = -(sig * sig) * jnp.log(m)

    cols = jax.lax.broadcasted_iota(jnp.int32, x.shape, 1)
    mask = (cols == lbl).astype(x.dtype)
    w = 0.75 - 0.5 * mask
    partial = jnp.sum(w * f, axis=(0, 1), keepdims=True)

    @pl.when(i == 0)
    def _():
        out_ref[...] = jnp.zeros_like(out_ref)

    out_ref[...] += partial


def kernel(cls_pred, cls_label):
    N, C = cls_pred.shape
    R = BLOCK_ROWS
    G = N // R
    lbl3 = cls_label.astype(jnp.int32).reshape(G, R, 1)

    total = pl.pallas_call(
        _loss_block_kernel,
        grid=(G,),
        in_specs=[
            pl.BlockSpec((R, C), lambda i: (i, 0)),
            pl.BlockSpec((1, R, 1), lambda i: (i, 0, 0)),
        ],
        out_specs=pl.BlockSpec((1, 1), lambda i: (0, 0)),
        out_shape=jax.ShapeDtypeStruct((1, 1), jnp.float32),
    )(cls_pred, lbl3)

    return (LOSS_WEIGHT / (N * C)) * total[0, 0]


# speedup vs baseline: 3.7912x; 1.0533x over previous
import functools

import jax
import jax.numpy as jnp
from jax.experimental import pallas as pl
from jax.experimental.pallas import tpu as pltpu

ALPHA = 0.25
LOSS_WEIGHT = 1.0

BLOCK_ROWS = 10000


def _loss_block_kernel(x_ref, lbl_ref, out_ref):
    i = pl.program_id(0)
    x = x_ref[...]
    lbl = lbl_ref[0]

    th = jnp.tanh(0.5 * x)
    sig = 0.5 + 0.5 * th
    m = 0.5 - 0.5 * th
    f = -(sig * sig) * jnp.log(m)

    cols = jax.lax.broadcasted_iota(jnp.int32, x.shape, 1)
    mask = (cols == lbl).astype(x.dtype)
    w = 0.75 - 0.5 * mask
    partial = jnp.sum(w * f, axis=(0, 1), keepdims=True)

    @pl.when(i == 0)
    def _():
        out_ref[...] = jnp.zeros_like(out_ref)

    out_ref[...] += partial


def kernel(cls_pred, cls_label):
    N, C = cls_pred.shape
    R = BLOCK_ROWS
    G = N // R
    lbl3 = cls_label.astype(jnp.int32).reshape(G, R, 1)

    total = pl.pallas_call(
        _loss_block_kernel,
        grid=(G,),
        in_specs=[
            pl.BlockSpec((R, C), lambda i: (i, 0)),
            pl.BlockSpec((1, R, 1), lambda i: (i, 0, 0)),
        ],
        out_specs=pl.BlockSpec((1, 1), lambda i: (0, 0)),
        out_shape=jax.ShapeDtypeStruct((1, 1), jnp.float32),
    )(cls_pred, lbl3)

    return (LOSS_WEIGHT / (N * C)) * total[0, 0]
